# Initial kernel scaffold; baseline (speedup 1.0000x reference)
import jax
import jax.numpy as jnp
from jax.experimental import pallas as pl


def kernel(x, edge_index, W1, a1s, a1d, b1, W2, a2s, a2d, b2, W3, a3s, a3d, b3):
    raise NotImplementedError("write your pallas kernel here")


# speedup vs baseline: 64.8920x; 64.8920x over previous
import functools

import jax
import jax.numpy as jnp
from jax import lax
from jax.experimental import pallas as pl
from jax.experimental.pallas import tpu as pltpu
from jax.experimental.pallas import tpu_sc as plsc

N = 10000
E = 320000
PADR = 112
NPAD = N + PADR
NC, NS, L = 2, 16, 16
NW = NC * NS
B = 64
ETOT = E + N
EPW = -(-ETOT // (NW * B)) * B
EPAD = EPW * NW
NBLK = EPW // B
RPT = NPAD // NS
TPAD = 10112
TRT = TPAD // NS
TW = 80
FP = 72


def _att_out(xw, asrc, adst, t_ref, m_ref, H):
    t_ref[...] = jnp.concatenate([xw, asrc[:, :8], adst[:, :8]], axis=1)
    m = (jnp.max(asrc, axis=0, keepdims=True)
         + jnp.max(adst, axis=0, keepdims=True))
    m = jnp.where(m > 0, m, 0.2 * m)
    m = jnp.roll(m, 8, axis=1)
    lanes = lax.broadcasted_iota(jnp.int32, (1, L), 1)
    m_ref[...] = jnp.where((lanes >= 8) & (lanes < 8 + H), m, 1e30)


def _dense_body(x_ref, wt_ref, as_ref, ad_ref, t_ref, m_ref, *, H):
    xw = jnp.dot(x_ref[...], wt_ref[...], preferred_element_type=jnp.float32)
    asrc = jnp.dot(xw, as_ref[...], preferred_element_type=jnp.float32)
    adst = jnp.dot(xw, ad_ref[...], preferred_element_type=jnp.float32)
    _att_out(xw, asrc, adst, t_ref, m_ref, H)


def _norm_in(acc_ref, p_ref, b_ref):
    a0 = acc_ref[0, :N, :]
    a1 = acc_ref[1, :N, :]
    s = a0[:, :64] + a1[:, :64]
    den = a0[:, 64:72] + a1[:, 64:72]
    denb = jnp.dot(den, p_ref[...], preferred_element_type=jnp.float32)
    return s / (denb + 1e-16) + b_ref[...]


def _dense_mid_body(acc_ref, p_ref, b_ref, wt_ref, as_ref, ad_ref,
                    t_ref, m_ref, *, H):
    h = _norm_in(acc_ref, p_ref, b_ref)
    h = jnp.where(h > 0, h, jnp.exp(h) - 1.0)
    xw = jnp.dot(h, wt_ref[...], preferred_element_type=jnp.float32)
    asrc = jnp.dot(xw, as_ref[...], preferred_element_type=jnp.float32)
    adst = jnp.dot(xw, ad_ref[...], preferred_element_type=jnp.float32)
    _att_out(xw, asrc, adst, t_ref, m_ref, H)


def _dense3_body(acc_ref, p_ref, b_ref, wt_ref, as_ref, ad_ref,
                 ta_ref, tb_ref, m_ref):
    h = _norm_in(acc_ref, p_ref, b_ref)
    h = jnp.where(h > 0, h, jnp.exp(h) - 1.0)
    xw = jnp.dot(h, wt_ref[...], preferred_element_type=jnp.float32)
    asrc = jnp.dot(xw, as_ref[...], preferred_element_type=jnp.float32)
    adst = jnp.dot(xw, ad_ref[...], preferred_element_type=jnp.float32)
    ta_ref[...] = jnp.concatenate([xw[:, :64], asrc[:, :8], adst[:, :8]],
                                  axis=1)
    tb_ref[...] = jnp.concatenate([xw[:, 64:], asrc[:, :8], adst[:, :8]],
                                  axis=1)
    m = (jnp.max(asrc, axis=0, keepdims=True)
         + jnp.max(adst, axis=0, keepdims=True))
    m = jnp.where(m > 0, m, 0.2 * m)
    m = jnp.roll(m, 8, axis=1)
    lanes = lax.broadcasted_iota(jnp.int32, (1, L), 1)
    m_ref[...] = jnp.where(lanes == 8, m, 1e30)


def _final_body(acca_ref, accb_ref, p_ref, b_ref, x_ref, o_ref):
    z = jnp.zeros((1, 64), jnp.float32)
    ha = _norm_in(acca_ref, p_ref, z)
    hb = _norm_in(accb_ref, p_ref, z)
    o_ref[...] = jnp.concatenate([ha, hb], axis=1) + b_ref[...] + x_ref[...]


def _tc_dense_first(x, wt, as16, ad16, H):
    return pl.pallas_call(
        functools.partial(_dense_body, H=H),
        out_shape=[
            jax.ShapeDtypeStruct((N, TW), jnp.float32),
            jax.ShapeDtypeStruct((1, L), jnp.float32),
        ],
    )(x, wt, as16, ad16)


def _tc_dense_mid(acc, p, brow, wt, as16, ad16, H):
    return pl.pallas_call(
        functools.partial(_dense_mid_body, H=H),
        out_shape=[
            jax.ShapeDtypeStruct((N, TW), jnp.float32),
            jax.ShapeDtypeStruct((1, L), jnp.float32),
        ],
    )(acc, p, brow, wt, as16, ad16)


def _tc_dense3(acc, p, brow, wt, as16, ad16):
    return pl.pallas_call(
        _dense3_body,
        out_shape=[
            jax.ShapeDtypeStruct((N, TW), jnp.float32),
            jax.ShapeDtypeStruct((N, TW), jnp.float32),
            jax.ShapeDtypeStruct((1, L), jnp.float32),
        ],
    )(acc, p, brow, wt, as16, ad16)


def _tc_final(acca, accb, p3, b3row, x):
    return pl.pallas_call(
        _final_body,
        out_shape=jax.ShapeDtypeStruct((N, 128), jnp.float32),
    )(acca, accb, p3, b3row, x)


def _splat(v, i):
    return jnp.broadcast_to(v[i], (16,))


def _edge_body(srcs, dsts, t_hbm, m_hbm, out_hbm,
               acc, t_sp, srcv, dstv, rows_s, rows_d, msg, m16v,
               sem_s, sem_d, *, C):
    cid = lax.axis_index("c")
    sid = lax.axis_index("s")
    wid = sid * NC + cid
    iot = lax.iota(jnp.int32, 16)
    roti = iot ^ 8

    pltpu.sync_copy(m_hbm, m16v)

    def _zrow(r, _):
        for off in (0, 16, 32, 48, 56):
            msg[r, pl.ds(off, 16)] = jnp.zeros((16,), jnp.float32)
        return 0
    lax.fori_loop(0, B, _zrow, 0)
    row0 = sid * RPT
    chunks = [(j * B, B) for j in range(RPT // B)]
    if RPT % B:
        chunks.append(((RPT // B) * B, RPT % B))
    for off, sz in chunks:
        r = row0 + off
        pltpu.sync_copy(msg.at[pl.ds(0, sz)], acc.at[pl.ds(r, sz)])
        rt = sid * TRT + off
        pltpu.sync_copy(t_hbm.at[pl.ds(rt, sz)], rows_s.at[pl.ds(0, sz)])
        pltpu.sync_copy(rows_s.at[pl.ds(0, sz)], t_sp.at[pl.ds(rt, sz)])
    plsc.subcore_barrier()

    mv = m16v[...]


    def _blk(blk, _):
        base = wid * EPW + blk * B
        pltpu.sync_copy(srcs.at[pl.ds(base, B)], srcv)
        pltpu.sync_copy(dsts.at[pl.ds(base, B)], dstv)
        cp_s = pltpu.async_copy(t_sp.at[srcv], rows_s, sem_s)
        cp_d = pltpu.async_copy(t_sp.at[dstv], rows_d, sem_d)
        cp_s.wait()
        cp_d.wait()

        def _edge(e, _):
            sv = rows_s[e, pl.ds(56, 16)]
            av = rows_d[e, pl.ds(64, 16)]
            al = sv + av
            al = jnp.where(al > 0, al, 0.2 * al)
            ex = jnp.exp(al - mv)
            for k in range(4):
                if C == 8:
                    exb = jnp.where(iot < 8, _splat(ex, 8 + 2 * k),
                                    _splat(ex, 9 + 2 * k))
                else:
                    exb = _splat(ex, 8)
                msg[e, pl.ds(16 * k, 16)] = rows_s[e, pl.ds(16 * k, 16)] * exb
            tail = sv * (_splat(ex, 15) if C == 8 else _splat(ex, 8))
            msg[e, pl.ds(56, 16)] = jnp.where(iot < 8, tail, ex)
            return 0
        lax.fori_loop(0, B, _edge, 0)
        pltpu.sync_copy(msg, acc.at[dstv], add=True)
        return 0
    lax.fori_loop(0, NBLK, _blk, 0)
    plsc.subcore_barrier()

    for off, sz in chunks:
        r = row0 + off
        pltpu.sync_copy(acc.at[pl.ds(r, sz)], msg.at[pl.ds(0, sz)])
        pltpu.sync_copy(msg.at[pl.ds(0, sz)], out_hbm.at[cid, pl.ds(r, sz)])


def _sc_edge(srcs, dsts, t, m16, C):
    mesh = plsc.VectorSubcoreMesh(core_axis_name="c", subcore_axis_name="s")
    return pl.kernel(
        functools.partial(_edge_body, C=C),
        mesh=mesh,
        out_type=jax.ShapeDtypeStruct((NC, NPAD, FP), jnp.float32),
        scratch_types=[
            pltpu.VMEM_SHARED((NPAD, FP), jnp.float32),
            pltpu.VMEM_SHARED((TPAD, TW), jnp.float32),
            pltpu.VMEM((B,), jnp.int32),
            pltpu.VMEM((B,), jnp.int32),
            pltpu.VMEM((B, TW), jnp.float32),
            pltpu.VMEM((B, TW), jnp.float32),
            pltpu.VMEM((B, FP), jnp.float32),
            pltpu.VMEM((16,), jnp.float32),
            pltpu.SemaphoreType.DMA,
            pltpu.SemaphoreType.DMA,
        ],
    )(srcs, dsts, t, m16)


def _block_diag_att(a, H, C):
    a2 = a.reshape(H, C)
    m = (jnp.eye(H, dtype=jnp.float32)[:, None, :] * a2[:, :, None])
    m = m.reshape(H * C, H)
    return jnp.concatenate([m, jnp.zeros((H * C, 16 - H), jnp.float32)], axis=1)


def _head_expand(H, C):
    p = jnp.repeat(jnp.eye(H, dtype=jnp.float32), C, axis=1)
    return jnp.concatenate([p, jnp.zeros((8 - H, H * C), jnp.float32)], axis=0)


def kernel(x, edge_index, W1, a1s, a1d, b1, W2, a2s, a2d, b2,
           W3, a3s, a3d, b3):
    npad_e = EPAD - ETOT
    loops = jnp.arange(N, dtype=jnp.int32)
    pad_src = jnp.arange(npad_e, dtype=jnp.int32) % N
    pad_dst = N + (jnp.arange(npad_e, dtype=jnp.int32) % PADR)
    srcs = jnp.concatenate([edge_index[0].astype(jnp.int32), loops, pad_src])
    dsts = jnp.concatenate([edge_index[1].astype(jnp.int32), loops, pad_dst])

    trows = jnp.zeros((TPAD - N, TW), jnp.float32)
    tp = lambda t: jnp.concatenate([t, trows])

    t1, m1 = _tc_dense_first(
        x, W1.T, _block_diag_att(a1s, 8, 8), _block_diag_att(a1d, 8, 8), 8)
    acc1 = _sc_edge(srcs, dsts, tp(t1), m1[0], 8)

    p8 = _head_expand(8, 8)
    t2, m2 = _tc_dense_mid(
        acc1, p8, b1.reshape(1, 64), W2.T,
        _block_diag_att(a2s, 8, 8), _block_diag_att(a2d, 8, 8), 8)
    acc2 = _sc_edge(srcs, dsts, tp(t2), m2[0], 8)

    t3a, t3b, m3 = _tc_dense3(
        acc2, p8, b2.reshape(1, 64), W3.T,
        _block_diag_att(a3s, 1, 128), _block_diag_att(a3d, 1, 128))
    acc3a = _sc_edge(srcs, dsts, tp(t3a), m3[0], 128)
    acc3b = _sc_edge(srcs, dsts, tp(t3b), m3[0], 128)

    return _tc_final(acc3a, acc3b, _head_expand(1, 64), b3.reshape(1, 128), x)


# speedup vs baseline: 65.3710x; 1.0074x over previous
import functools

import jax
import jax.numpy as jnp
from jax import lax
from jax.experimental import pallas as pl
from jax.experimental.pallas import tpu as pltpu
from jax.experimental.pallas import tpu_sc as plsc

N = 10000
E = 320000
PADR = 112
NPAD = N + PADR
NC, NS, L = 2, 16, 16
NW = NC * NS
B = 64
ETOT = E + N
EPW = -(-ETOT // (NW * B)) * B
EPAD = EPW * NW
NBLK = EPW // B
RPT = NPAD // NS
TPAD = 10112
TRT = TPAD // NS
TW = 80
FP = 72


def _att_out(xw, asrc, adst, t_ref, m_ref, H):
    t_ref[...] = jnp.concatenate([xw, asrc[:, :8], adst[:, :8]], axis=1)
    m = (jnp.max(asrc, axis=0, keepdims=True)
         + jnp.max(adst, axis=0, keepdims=True))
    m = jnp.where(m > 0, m, 0.2 * m)
    m = jnp.roll(m, 8, axis=1)
    lanes = lax.broadcasted_iota(jnp.int32, (1, L), 1)
    m_ref[...] = jnp.where((lanes >= 8) & (lanes < 8 + H), m, 1e30)


def _dense_body(x_ref, wt_ref, as_ref, ad_ref, t_ref, m_ref, *, H):
    xw = jnp.dot(x_ref[...], wt_ref[...], preferred_element_type=jnp.float32)
    asrc = jnp.dot(xw, as_ref[...], preferred_element_type=jnp.float32)
    adst = jnp.dot(xw, ad_ref[...], preferred_element_type=jnp.float32)
    _att_out(xw, asrc, adst, t_ref, m_ref, H)


def _norm_in(acc_ref, p_ref, b_ref):
    a0 = acc_ref[0, :N, :]
    a1 = acc_ref[1, :N, :]
    s = a0[:, :64] + a1[:, :64]
    den = a0[:, 64:72] + a1[:, 64:72]
    denb = jnp.dot(den, p_ref[...], preferred_element_type=jnp.float32)
    return s / (denb + 1e-16) + b_ref[...]


def _dense_mid_body(acc_ref, p_ref, b_ref, wt_ref, as_ref, ad_ref,
                    t_ref, m_ref, *, H):
    h = _norm_in(acc_ref, p_ref, b_ref)
    h = jnp.where(h > 0, h, jnp.exp(h) - 1.0)
    xw = jnp.dot(h, wt_ref[...], preferred_element_type=jnp.float32)
    asrc = jnp.dot(xw, as_ref[...], preferred_element_type=jnp.float32)
    adst = jnp.dot(xw, ad_ref[...], preferred_element_type=jnp.float32)
    _att_out(xw, asrc, adst, t_ref, m_ref, H)


def _dense3_body(acc_ref, p_ref, b_ref, wt_ref, as_ref, ad_ref,
                 ta_ref, tb_ref, m_ref):
    h = _norm_in(acc_ref, p_ref, b_ref)
    h = jnp.where(h > 0, h, jnp.exp(h) - 1.0)
    xw = jnp.dot(h, wt_ref[...], preferred_element_type=jnp.float32)
    asrc = jnp.dot(xw, as_ref[...], preferred_element_type=jnp.float32)
    adst = jnp.dot(xw, ad_ref[...], preferred_element_type=jnp.float32)
    ta_ref[...] = jnp.concatenate([xw[:, :64], asrc[:, :8], adst[:, :8]],
                                  axis=1)
    tb_ref[...] = jnp.concatenate([xw[:, 64:], asrc[:, :8], adst[:, :8]],
                                  axis=1)
    m = (jnp.max(asrc, axis=0, keepdims=True)
         + jnp.max(adst, axis=0, keepdims=True))
    m = jnp.where(m > 0, m, 0.2 * m)
    m = jnp.roll(m, 8, axis=1)
    lanes = lax.broadcasted_iota(jnp.int32, (1, L), 1)
    m_ref[...] = jnp.where(lanes == 8, m, 1e30)


def _final_body(acca_ref, accb_ref, p_ref, b_ref, x_ref, o_ref):
    z = jnp.zeros((1, 64), jnp.float32)
    ha = _norm_in(acca_ref, p_ref, z)
    hb = _norm_in(accb_ref, p_ref, z)
    o_ref[...] = jnp.concatenate([ha, hb], axis=1) + b_ref[...] + x_ref[...]


def _tc_dense_first(x, wt, as16, ad16, H):
    return pl.pallas_call(
        functools.partial(_dense_body, H=H),
        out_shape=[
            jax.ShapeDtypeStruct((N, TW), jnp.float32),
            jax.ShapeDtypeStruct((1, L), jnp.float32),
        ],
    )(x, wt, as16, ad16)


def _tc_dense_mid(acc, p, brow, wt, as16, ad16, H):
    return pl.pallas_call(
        functools.partial(_dense_mid_body, H=H),
        out_shape=[
            jax.ShapeDtypeStruct((N, TW), jnp.float32),
            jax.ShapeDtypeStruct((1, L), jnp.float32),
        ],
    )(acc, p, brow, wt, as16, ad16)


def _tc_dense3(acc, p, brow, wt, as16, ad16):
    return pl.pallas_call(
        _dense3_body,
        out_shape=[
            jax.ShapeDtypeStruct((N, TW), jnp.float32),
            jax.ShapeDtypeStruct((N, TW), jnp.float32),
            jax.ShapeDtypeStruct((1, L), jnp.float32),
        ],
    )(acc, p, brow, wt, as16, ad16)


def _tc_final(acca, accb, p3, b3row, x):
    return pl.pallas_call(
        _final_body,
        out_shape=jax.ShapeDtypeStruct((N, 128), jnp.float32),
    )(acca, accb, p3, b3row, x)


def _splat(v, i):
    return jnp.broadcast_to(v[i], (16,))


def _edge_body(srcs, dsts, t_hbm, m_hbm, out_hbm,
               acc, t_sp, srcv, dstv, rows_s, rows_d, msg, m16v,
               sem_s, sem_d, *, C):
    cid = lax.axis_index("c")
    sid = lax.axis_index("s")
    wid = sid * NC + cid
    iot = lax.iota(jnp.int32, 16)
    roti = iot ^ 8

    pltpu.sync_copy(m_hbm, m16v)

    def _zrow(r, _):
        for off in (0, 16, 32, 48, 56):
            msg[r, pl.ds(off, 16)] = jnp.zeros((16,), jnp.float32)
        return 0
    lax.fori_loop(0, B, _zrow, 0)
    row0 = sid * RPT
    chunks = [(j * B, B) for j in range(RPT // B)]
    if RPT % B:
        chunks.append(((RPT // B) * B, RPT % B))
    for off, sz in chunks:
        r = row0 + off
        pltpu.sync_copy(msg.at[pl.ds(0, sz)], acc.at[pl.ds(r, sz)])
        rt = sid * TRT + off
        pltpu.sync_copy(t_hbm.at[pl.ds(rt, sz)], rows_s.at[pl.ds(0, sz)])
        pltpu.sync_copy(rows_s.at[pl.ds(0, sz)], t_sp.at[pl.ds(rt, sz)])
    plsc.subcore_barrier()

    mv = m16v[...]


    def _blk(blk, _):
        base = wid * EPW + blk * B
        pltpu.sync_copy(srcs.at[pl.ds(base, B)], srcv)
        pltpu.sync_copy(dsts.at[pl.ds(base, B)], dstv)
        cp_s = pltpu.async_copy(t_sp.at[srcv], rows_s, sem_s)
        cp_d = pltpu.async_copy(t_sp.at[dstv], rows_d, sem_d)
        cp_s.wait()
        cp_d.wait()

        def _edge(e, _):
            sv = rows_s[e, pl.ds(56, 16)]
            av = rows_d[e, pl.ds(64, 16)]
            al = sv + av
            al = jnp.where(al > 0, al, 0.2 * al)
            ex = jnp.exp(al - mv)
            for k in range(4):
                if C == 8:
                    exb = jnp.where(iot < 8, _splat(ex, 8 + 2 * k),
                                    _splat(ex, 9 + 2 * k))
                else:
                    exb = _splat(ex, 8)
                msg[e, pl.ds(16 * k, 16)] = rows_s[e, pl.ds(16 * k, 16)] * exb
            tail = sv * (_splat(ex, 15) if C == 8 else _splat(ex, 8))
            msg[e, pl.ds(56, 16)] = jnp.where(iot < 8, tail, ex)
            return 0
        def _edge2(i, _):
            _edge(2 * i, 0)
            _edge(2 * i + 1, 0)
            return 0
        lax.fori_loop(0, B // 2, _edge2, 0)
        pltpu.sync_copy(msg, acc.at[dstv], add=True)
        return 0
    lax.fori_loop(0, NBLK, _blk, 0)
    plsc.subcore_barrier()

    for off, sz in chunks:
        r = row0 + off
        pltpu.sync_copy(acc.at[pl.ds(r, sz)], msg.at[pl.ds(0, sz)])
        pltpu.sync_copy(msg.at[pl.ds(0, sz)], out_hbm.at[cid, pl.ds(r, sz)])


def _sc_edge(srcs, dsts, t, m16, C):
    mesh = plsc.VectorSubcoreMesh(core_axis_name="c", subcore_axis_name="s")
    return pl.kernel(
        functools.partial(_edge_body, C=C),
        mesh=mesh,
        out_type=jax.ShapeDtypeStruct((NC, NPAD, FP), jnp.float32),
        scratch_types=[
            pltpu.VMEM_SHARED((NPAD, FP), jnp.float32),
            pltpu.VMEM_SHARED((TPAD, TW), jnp.float32),
            pltpu.VMEM((B,), jnp.int32),
            pltpu.VMEM((B,), jnp.int32),
            pltpu.VMEM((B, TW), jnp.float32),
            pltpu.VMEM((B, TW), jnp.float32),
            pltpu.VMEM((B, FP), jnp.float32),
            pltpu.VMEM((16,), jnp.float32),
            pltpu.SemaphoreType.DMA,
            pltpu.SemaphoreType.DMA,
        ],
    )(srcs, dsts, t, m16)


def _block_diag_att(a, H, C):
    a2 = a.reshape(H, C)
    m = (jnp.eye(H, dtype=jnp.float32)[:, None, :] * a2[:, :, None])
    m = m.reshape(H * C, H)
    return jnp.concatenate([m, jnp.zeros((H * C, 16 - H), jnp.float32)], axis=1)


def _head_expand(H, C):
    p = jnp.repeat(jnp.eye(H, dtype=jnp.float32), C, axis=1)
    return jnp.concatenate([p, jnp.zeros((8 - H, H * C), jnp.float32)], axis=0)


def kernel(x, edge_index, W1, a1s, a1d, b1, W2, a2s, a2d, b2,
           W3, a3s, a3d, b3):
    npad_e = EPAD - ETOT
    loops = jnp.arange(N, dtype=jnp.int32)
    pad_src = jnp.arange(npad_e, dtype=jnp.int32) % N
    pad_dst = N + (jnp.arange(npad_e, dtype=jnp.int32) % PADR)
    srcs = jnp.concatenate([edge_index[0].astype(jnp.int32), loops, pad_src])
    dsts = jnp.concatenate([edge_index[1].astype(jnp.int32), loops, pad_dst])

    trows = jnp.zeros((TPAD - N, TW), jnp.float32)
    tp = lambda t: jnp.concatenate([t, trows])

    t1, m1 = _tc_dense_first(
        x, W1.T, _block_diag_att(a1s, 8, 8), _block_diag_att(a1d, 8, 8), 8)
    acc1 = _sc_edge(srcs, dsts, tp(t1), m1[0], 8)

    p8 = _head_expand(8, 8)
    t2, m2 = _tc_dense_mid(
        acc1, p8, b1.reshape(1, 64), W2.T,
        _block_diag_att(a2s, 8, 8), _block_diag_att(a2d, 8, 8), 8)
    acc2 = _sc_edge(srcs, dsts, tp(t2), m2[0], 8)

    t3a, t3b, m3 = _tc_dense3(
        acc2, p8, b2.reshape(1, 64), W3.T,
        _block_diag_att(a3s, 1, 128), _block_diag_att(a3d, 1, 128))
    acc3a = _sc_edge(srcs, dsts, tp(t3a), m3[0], 128)
    acc3b = _sc_edge(srcs, dsts, tp(t3b), m3[0], 128)

    return _tc_final(acc3a, acc3b, _head_expand(1, 64), b3.reshape(1, 128), x)


# speedup vs baseline: 65.4347x; 1.0010x over previous
import functools

import jax
import jax.numpy as jnp
from jax import lax
from jax.experimental import pallas as pl
from jax.experimental.pallas import tpu as pltpu
from jax.experimental.pallas import tpu_sc as plsc

N = 10000
E = 320000
PADR = 112
NPAD = N + PADR
NC, NS, L = 2, 16, 16
NW = NC * NS
B = 64
ETOT = E + N
EPW = -(-ETOT // (NW * B)) * B
EPAD = EPW * NW
NBLK = EPW // B
RPT = NPAD // NS
TPAD = 10112
TRT = TPAD // NS
TW = 80
FP = 72


def _att_out(xw, asrc, adst, t_ref, m_ref, H):
    t_ref[...] = jnp.concatenate([xw, asrc[:, :8], adst[:, :8]], axis=1)
    m = (jnp.max(asrc, axis=0, keepdims=True)
         + jnp.max(adst, axis=0, keepdims=True))
    m = jnp.where(m > 0, m, 0.2 * m)
    m = jnp.roll(m, 8, axis=1)
    lanes = lax.broadcasted_iota(jnp.int32, (1, L), 1)
    m_ref[...] = jnp.where((lanes >= 8) & (lanes < 8 + H), m, 1e30)


def _dense_body(x_ref, wt_ref, as_ref, ad_ref, t_ref, m_ref, *, H):
    xw = jnp.dot(x_ref[...], wt_ref[...], preferred_element_type=jnp.float32)
    asrc = jnp.dot(xw, as_ref[...], preferred_element_type=jnp.float32)
    adst = jnp.dot(xw, ad_ref[...], preferred_element_type=jnp.float32)
    _att_out(xw, asrc, adst, t_ref, m_ref, H)


def _norm_in(acc_ref, p_ref, b_ref):
    a0 = acc_ref[0, :N, :]
    a1 = acc_ref[1, :N, :]
    s = a0[:, :64] + a1[:, :64]
    den = a0[:, 64:72] + a1[:, 64:72]
    denb = jnp.dot(den, p_ref[...], preferred_element_type=jnp.float32)
    return s / (denb + 1e-16) + b_ref[...]


def _dense_mid_body(acc_ref, p_ref, b_ref, wt_ref, as_ref, ad_ref,
                    t_ref, m_ref, *, H):
    h = _norm_in(acc_ref, p_ref, b_ref)
    h = jnp.where(h > 0, h, jnp.exp(h) - 1.0)
    xw = jnp.dot(h, wt_ref[...], preferred_element_type=jnp.float32)
    asrc = jnp.dot(xw, as_ref[...], preferred_element_type=jnp.float32)
    adst = jnp.dot(xw, ad_ref[...], preferred_element_type=jnp.float32)
    _att_out(xw, asrc, adst, t_ref, m_ref, H)


def _dense3_body(acc_ref, p_ref, b_ref, wt_ref, as_ref, ad_ref,
                 ta_ref, tb_ref, m_ref):
    h = _norm_in(acc_ref, p_ref, b_ref)
    h = jnp.where(h > 0, h, jnp.exp(h) - 1.0)
    xw = jnp.dot(h, wt_ref[...], preferred_element_type=jnp.float32)
    asrc = jnp.dot(xw, as_ref[...], preferred_element_type=jnp.float32)
    adst = jnp.dot(xw, ad_ref[...], preferred_element_type=jnp.float32)
    ta_ref[...] = jnp.concatenate([xw[:, :64], asrc[:, :8], adst[:, :8]],
                                  axis=1)
    tb_ref[...] = jnp.concatenate([xw[:, 64:], asrc[:, :8], adst[:, :8]],
                                  axis=1)
    m = (jnp.max(asrc, axis=0, keepdims=True)
         + jnp.max(adst, axis=0, keepdims=True))
    m = jnp.where(m > 0, m, 0.2 * m)
    m = jnp.roll(m, 8, axis=1)
    lanes = lax.broadcasted_iota(jnp.int32, (1, L), 1)
    m_ref[...] = jnp.where(lanes == 8, m, 1e30)


def _final_body(acca_ref, accb_ref, p_ref, b_ref, x_ref, o_ref):
    z = jnp.zeros((1, 64), jnp.float32)
    ha = _norm_in(acca_ref, p_ref, z)
    hb = _norm_in(accb_ref, p_ref, z)
    o_ref[...] = jnp.concatenate([ha, hb], axis=1) + b_ref[...] + x_ref[...]


def _tc_dense_first(x, wt, as16, ad16, H):
    return pl.pallas_call(
        functools.partial(_dense_body, H=H),
        out_shape=[
            jax.ShapeDtypeStruct((N, TW), jnp.float32),
            jax.ShapeDtypeStruct((1, L), jnp.float32),
        ],
    )(x, wt, as16, ad16)


def _tc_dense_mid(acc, p, brow, wt, as16, ad16, H):
    return pl.pallas_call(
        functools.partial(_dense_mid_body, H=H),
        out_shape=[
            jax.ShapeDtypeStruct((N, TW), jnp.float32),
            jax.ShapeDtypeStruct((1, L), jnp.float32),
        ],
    )(acc, p, brow, wt, as16, ad16)


def _tc_dense3(acc, p, brow, wt, as16, ad16):
    return pl.pallas_call(
        _dense3_body,
        out_shape=[
            jax.ShapeDtypeStruct((N, TW), jnp.float32),
            jax.ShapeDtypeStruct((N, TW), jnp.float32),
            jax.ShapeDtypeStruct((1, L), jnp.float32),
        ],
    )(acc, p, brow, wt, as16, ad16)


def _tc_final(acca, accb, p3, b3row, x):
    return pl.pallas_call(
        _final_body,
        out_shape=jax.ShapeDtypeStruct((N, 128), jnp.float32),
    )(acca, accb, p3, b3row, x)


def _splat(v, i):
    return jnp.broadcast_to(v[i], (16,))


def _edge_body(srcs, dsts, t_hbm, m_hbm, out_hbm,
               acc, t_sp, srcv, dstv, rows_s, rows_d, msg, m16v,
               sem_s, sem_d, *, C):
    cid = lax.axis_index("c")
    sid = lax.axis_index("s")
    wid = sid * NC + cid
    iot = lax.iota(jnp.int32, 16)

    pltpu.sync_copy(m_hbm, m16v)

    def _zrow(r, _):
        for off in (0, 16, 32, 48, 56):
            msg[r, pl.ds(off, 16)] = jnp.zeros((16,), jnp.float32)
        return 0
    lax.fori_loop(0, B, _zrow, 0)
    row0 = sid * RPT
    chunks = [(j * B, B) for j in range(RPT // B)]
    if RPT % B:
        chunks.append(((RPT // B) * B, RPT % B))
    for off, sz in chunks:
        r = row0 + off
        pltpu.sync_copy(msg.at[pl.ds(0, sz)], acc.at[pl.ds(r, sz)])
        rt = sid * TRT + off
        pltpu.sync_copy(t_hbm.at[pl.ds(rt, sz)], rows_s.at[pl.ds(0, sz)])
        pltpu.sync_copy(rows_s.at[pl.ds(0, sz)], t_sp.at[pl.ds(rt, sz)])
    plsc.subcore_barrier()

    mv = m16v[...]

    def _blk(blk, _):
        base = wid * EPW + blk * B
        pltpu.sync_copy(srcs.at[pl.ds(base, B)], srcv)
        pltpu.sync_copy(dsts.at[pl.ds(base, B)], dstv)
        cp_s = pltpu.async_copy(t_sp.at[srcv], rows_s, sem_s)
        cp_d = pltpu.async_copy(t_sp.at[dstv], rows_d, sem_d)
        cp_s.wait()
        cp_d.wait()

        def _edge(e, _):
            sv = rows_s[e, pl.ds(56, 16)]
            av = rows_d[e, pl.ds(64, 16)]
            al = sv + av
            al = jnp.where(al > 0, al, 0.2 * al)
            ex = jnp.exp(al - mv)
            for k in range(4):
                if C == 8:
                    exb = jnp.where(iot < 8, _splat(ex, 8 + 2 * k),
                                    _splat(ex, 9 + 2 * k))
                else:
                    exb = _splat(ex, 8)
                msg[e, pl.ds(16 * k, 16)] = rows_s[e, pl.ds(16 * k, 16)] * exb
            tail = sv * (_splat(ex, 15) if C == 8 else _splat(ex, 8))
            msg[e, pl.ds(56, 16)] = jnp.where(iot < 8, tail, ex)
            return 0
        def _edge2(i, _):
            _edge(2 * i, 0)
            _edge(2 * i + 1, 0)
            return 0
        lax.fori_loop(0, B // 2, _edge2, 0)
        pltpu.sync_copy(msg, acc.at[dstv], add=True)
        return 0
    lax.fori_loop(0, NBLK, _blk, 0)
    plsc.subcore_barrier()

    for off, sz in chunks:
        r = row0 + off
        pltpu.sync_copy(acc.at[pl.ds(r, sz)], msg.at[pl.ds(0, sz)])
        pltpu.sync_copy(msg.at[pl.ds(0, sz)], out_hbm.at[cid, pl.ds(r, sz)])


def _sc_edge(srcs, dsts, t, m16, C):
    mesh = plsc.VectorSubcoreMesh(core_axis_name="c", subcore_axis_name="s")
    return pl.kernel(
        functools.partial(_edge_body, C=C),
        mesh=mesh,
        out_type=jax.ShapeDtypeStruct((NC, NPAD, FP), jnp.float32),
        scratch_types=[
            pltpu.VMEM_SHARED((NPAD, FP), jnp.float32),
            pltpu.VMEM_SHARED((TPAD, TW), jnp.float32),
            pltpu.VMEM((B,), jnp.int32),
            pltpu.VMEM((B,), jnp.int32),
            pltpu.VMEM((B, TW), jnp.float32),
            pltpu.VMEM((B, TW), jnp.float32),
            pltpu.VMEM((B, FP), jnp.float32),
            pltpu.VMEM((16,), jnp.float32),
            pltpu.SemaphoreType.DMA,
            pltpu.SemaphoreType.DMA,
        ],
    )(srcs, dsts, t, m16)


def _block_diag_att(a, H, C):
    a2 = a.reshape(H, C)
    m = (jnp.eye(H, dtype=jnp.float32)[:, None, :] * a2[:, :, None])
    m = m.reshape(H * C, H)
    return jnp.concatenate([m, jnp.zeros((H * C, 16 - H), jnp.float32)], axis=1)


def _head_expand(H, C):
    p = jnp.repeat(jnp.eye(H, dtype=jnp.float32), C, axis=1)
    return jnp.concatenate([p, jnp.zeros((8 - H, H * C), jnp.float32)], axis=0)


def kernel(x, edge_index, W1, a1s, a1d, b1, W2, a2s, a2d, b2,
           W3, a3s, a3d, b3):
    npad_e = EPAD - ETOT
    loops = jnp.arange(N, dtype=jnp.int32)
    pad_src = jnp.arange(npad_e, dtype=jnp.int32) % N
    pad_dst = N + (jnp.arange(npad_e, dtype=jnp.int32) % PADR)
    srcs = jnp.concatenate([edge_index[0].astype(jnp.int32), loops, pad_src])
    dsts = jnp.concatenate([edge_index[1].astype(jnp.int32), loops, pad_dst])

    trows = jnp.zeros((TPAD - N, TW), jnp.float32)
    tp = lambda t: jnp.concatenate([t, trows])

    t1, m1 = _tc_dense_first(
        x, W1.T, _block_diag_att(a1s, 8, 8), _block_diag_att(a1d, 8, 8), 8)
    acc1 = _sc_edge(srcs, dsts, tp(t1), m1[0], 8)

    p8 = _head_expand(8, 8)
    t2, m2 = _tc_dense_mid(
        acc1, p8, b1.reshape(1, 64), W2.T,
        _block_diag_att(a2s, 8, 8), _block_diag_att(a2d, 8, 8), 8)
    acc2 = _sc_edge(srcs, dsts, tp(t2), m2[0], 8)

    t3a, t3b, m3 = _tc_dense3(
        acc2, p8, b2.reshape(1, 64), W3.T,
        _block_diag_att(a3s, 1, 128), _block_diag_att(a3d, 1, 128))
    acc3a = _sc_edge(srcs, dsts, tp(t3a), m3[0], 128)
    acc3b = _sc_edge(srcs, dsts, tp(t3b), m3[0], 128)

    return _tc_final(acc3a, acc3b, _head_expand(1, 64), b3.reshape(1, 128), x)
